# baseline (speedup 1.0000x reference)
import functools

import jax
import jax.numpy as jnp
from jax import lax
from jax.experimental import pallas as pl
from jax.experimental.pallas import tpu as pltpu
from jax.experimental.pallas import tpu_sc as plsc

_BATCH = 16384
_DIM = 64
_GRP = 32


def _make_kernel(num_cores, num_subcores):
    nw = num_cores * num_subcores
    b_per_w = _BATCH // nw
    n2 = 2 * b_per_w
    n_grp = b_per_w // _GRP
    gt = _GRP // 8
    mesh = plsc.VectorSubcoreMesh(core_axis_name="c", subcore_axis_name="s")

    @functools.partial(
        pl.kernel,
        out_type=(
            jax.ShapeDtypeStruct((_BATCH, _DIM), jnp.float32),
            jax.ShapeDtypeStruct((_BATCH, _DIM), jnp.float32),
        ),
        mesh=mesh,
        scratch_types=[
            pltpu.VMEM((n2,), jnp.int32),
            pltpu.VMEM((2 * _GRP, 8, _DIM), jnp.float32),
            pltpu.VMEM((2, gt, 8, _DIM), jnp.float32),
            pltpu.SemaphoreType.DMA,
            pltpu.SemaphoreType.DMA,
        ],
    )
    def k(uidx_hbm, iidx_hbm, utab, itab, uout, iout,
          idx_v, slab_v, oring_v, sem, osem):
        wid = lax.axis_index("s") * num_cores + lax.axis_index("c")
        base = pl.multiple_of(wid * b_per_w, b_per_w)
        pltpu.sync_copy(uidx_hbm.at[pl.ds(base, b_per_w)],
                        idx_v.at[pl.ds(0, b_per_w)])
        pltpu.sync_copy(iidx_hbm.at[pl.ds(base, b_per_w)],
                        idx_v.at[pl.ds(b_per_w, b_per_w)])
        utab3 = utab.reshape(utab.shape[0] // 8, 8, _DIM)
        itab3 = itab.reshape(itab.shape[0] // 8, 8, _DIM)
        uout3 = uout.reshape(_BATCH // 8, 8, _DIM)
        iout3 = iout.reshape(_BATCH // 8, 8, _DIM)

        def issue_group(tab3, jbase, ring):
            for b in range(_GRP // 16):
                v = idx_v[pl.ds(jbase + b * 16, 16)]
                for u in range(16):
                    t = v[u] >> 3
                    pltpu.async_copy(
                        tab3.at[t], slab_v.at[ring + b * 16 + u], sem)

        def select_group(jbase, ring, oslot):
            for b in range(_GRP // 16):
                v = idx_v[pl.ds(jbase + b * 16, 16)]
                for u in range(16):
                    uu = b * 16 + u
                    rr = v[u] & 7
                    for d in range(_DIM // 16):
                        oring_v[oslot, uu >> 3, uu & 7,
                                pl.ds(d * 16, 16)] = (
                            slab_v[ring + uu, rr, pl.ds(d * 16, 16)])

        for half_id in range(2):
            tab3 = utab3 if half_id == 0 else itab3
            out3 = uout3 if half_id == 0 else iout3
            jb0 = half_id * b_per_w
            otile0 = wid * (b_per_w // 8)
            issue_group(tab3, jb0, 0)

            def body(g, _):
                ring = (g % 2) * _GRP
                oslot = g % 2

                @pl.when(g + 1 < n_grp)
                def _issue():
                    issue_group(tab3, jb0 + (g + 1) * _GRP,
                                ((g + 1) % 2) * _GRP)

                pltpu.make_async_copy(
                    tab3.at[pl.ds(0, _GRP)],
                    slab_v.at[pl.ds(ring, _GRP)], sem).wait()

                @pl.when(g >= 2)
                def _owait():
                    pltpu.make_async_copy(
                        oring_v.at[oslot], out3.at[pl.ds(0, gt)],
                        osem).wait()

                select_group(jb0 + g * _GRP, ring, oslot)
                pltpu.async_copy(oring_v.at[oslot],
                                 out3.at[pl.ds(otile0 + g * gt, gt)],
                                 osem)
                return _

            lax.fori_loop(0, n_grp, body, 0)
            for tail in range(2):
                pltpu.make_async_copy(
                    oring_v.at[tail], out3.at[pl.ds(0, gt)], osem).wait()

    return k


def kernel(user_indices, item_indices, user_table, item_table):
    info = plsc.get_sparse_core_info()
    k = _make_kernel(info.num_cores, info.num_subcores)
    uidx = user_indices.astype(jnp.int32)
    iidx = item_indices.astype(jnp.int32)
    return k(uidx, iidx, user_table, item_table)
